# baseline (speedup 1.0000x reference)
import functools

import jax
import jax.numpy as jnp
from jax import lax
from jax.experimental import pallas as pl
from jax.experimental.pallas import tpu as pltpu
from jax.experimental.pallas import tpu_sc as plsc

OUT_SIZE = 7
NB = OUT_SIZE * OUT_SIZE
C = 256
CV = C // 16
NC, NS = 2, 16
NW = NC * NS
RPW = 32
NROI = 1000
NROI_PAD = NW * RPW
CHUNK_BINS = 7
NCHUNK = NB // CHUNK_BINS
CHUNK_ROWS = CHUNK_BINS * 16
TOT_ROWS = 2 * (128 * 128 + 64 * 64 + 32 * 32 + 16 * 16)

_T1 = float((56.0 * (2.0 ** 1 - 1e-6)) ** 2)
_T2 = float((56.0 * (2.0 ** 2 - 1e-6)) ** 2)
_T3 = float((56.0 * (2.0 ** 3 - 1e-6)) ** 2)

_GATHER_DNUMS = lax.GatherDimensionNumbers(
    offset_dims=(), collapsed_slice_dims=(0,), start_index_map=(0,))


def _vreg_gather(vec, idx):
    return lax.gather(vec, idx[:, None], _GATHER_DNUMS, (1,),
                      mode=lax.GatherScatterMode.PROMISE_IN_BOUNDS)


def _sc_body(table, rb, rx1, ry1, rx2, ry2, out,
             rbv, rx1v, ry1v, rx2v, ry2v,
             idx_buf, w_buf, rows_buf, bins_buf, sem0, sem1):
    wid = lax.axis_index("c") * NS + lax.axis_index("s")
    base_roi = wid * RPW
    pltpu.sync_copy(rb.at[pl.ds(base_roi, RPW)], rbv)
    pltpu.sync_copy(rx1.at[pl.ds(base_roi, RPW)], rx1v)
    pltpu.sync_copy(ry1.at[pl.ds(base_roi, RPW)], ry1v)
    pltpu.sync_copy(rx2.at[pl.ds(base_roi, RPW)], rx2v)
    pltpu.sync_copy(ry2.at[pl.ds(base_roi, RPW)], ry2v)

    lane = jnp.arange(16, dtype=jnp.int32)
    dy = lane >> 3
    dx = (lane >> 2) & 1
    ty = (lane >> 1) & 1
    tx = lane & 1

    def roi_body(r, carry):
        half = r & 16
        ridx = jnp.full((16,), r & 15, jnp.int32)

        def bcast(ref):
            return _vreg_gather(ref[pl.ds(half, 16)], ridx)

        bb = bcast(rbv)
        xx1 = bcast(rx1v)
        yy1 = bcast(ry1v)
        xx2 = bcast(rx2v)
        yy2 = bcast(ry2v)

        area = (xx2 - xx1 + 1.0) * (yy2 - yy1 + 1.0)
        c1 = area >= _T1
        c2 = area >= _T2
        c3 = area >= _T3
        ss = jnp.where(c3, 0.03125, jnp.where(c2, 0.0625, jnp.where(c1, 0.125, 0.25)))
        wdim = jnp.where(c3, 16, jnp.where(c2, 32, jnp.where(c1, 64, 128)))
        basev = jnp.where(c3, 43008, jnp.where(c2, 40960, jnp.where(c1, 32768, 0)))
        basev = basev + bb.astype(jnp.int32) * (wdim * wdim)
        wm1 = wdim - 1
        wm1f = wm1.astype(jnp.float32)
        x1s = xx1 * ss
        y1s = yy1 * ss
        rw7 = jnp.maximum(xx2 * ss - x1s, 1.0) * (1.0 / OUT_SIZE)
        rh7 = jnp.maximum(yy2 * ss - y1s, 1.0) * (1.0 / OUT_SIZE)

        def row_body(i, _):
            def col_body(j, _):
                s_y = dy + 2 * i
                s_x = dx + 2 * j
                yy = y1s + (s_y.astype(jnp.float32) + 0.5) * 0.5 * rh7
                xx = x1s + (s_x.astype(jnp.float32) + 0.5) * 0.5 * rw7
                yy = jnp.clip(yy, 0.0, wm1f)
                xx = jnp.clip(xx, 0.0, wm1f)
                y0 = yy.astype(jnp.int32)
                x0 = xx.astype(jnp.int32)
                lyv = yy - y0.astype(jnp.float32)
                lxv = xx - x0.astype(jnp.float32)
                y1c = jnp.minimum(y0 + 1, wm1)
                x1c = jnp.minimum(x0 + 1, wm1)
                ysel = jnp.where(ty == 1, y1c, y0)
                xsel = jnp.where(tx == 1, x1c, x0)
                wyv = jnp.where(ty == 1, lyv, 1.0 - lyv)
                wxv = jnp.where(tx == 1, lxv, 1.0 - lxv)
                idx_buf[i, pl.ds(j * 16, 16)] = basev + ysel * wdim + xsel
                w_buf[i * OUT_SIZE + j, :] = wyv * wxv * 0.25
                return 0
            return lax.fori_loop(0, OUT_SIZE, col_body, 0)
        lax.fori_loop(0, OUT_SIZE, row_body, 0)

        sems = (sem0, sem1)

        def fire(cs):
            return pltpu.async_copy(table.at[idx_buf.at[cs]],
                                    rows_buf.at[cs % 2], sems[cs % 2])

        handles = [fire(0)]
        for cs in range(NCHUNK):
            handles[cs].wait()
            if cs + 1 < NCHUNK:
                handles.append(fire(cs + 1))

            def bin_body(bi, _, cs=cs):
                binid = cs * CHUNK_BINS + bi
                wv = w_buf[binid, :]
                acc = [jnp.zeros((16,), jnp.float32) for _ in range(CV)]
                for q in range(16):
                    wq = _vreg_gather(wv, jnp.full((16,), q, jnp.int32))
                    rrow = bi * 16 + q
                    for cc in range(CV):
                        acc[cc] = acc[cc] + wq * rows_buf[cs % 2, rrow,
                                                          pl.ds(cc * 16, 16)]
                for cc in range(CV):
                    bins_buf[binid, pl.ds(cc * 16, 16)] = acc[cc]
                return 0
            lax.fori_loop(0, CHUNK_BINS, bin_body, 0)

        rg = base_roi + r

        @pl.when(rg < NROI)
        def _():
            pltpu.sync_copy(bins_buf, out.at[rg])

        return carry

    lax.fori_loop(0, RPW, roi_body, 0)


_roi_align_sc = functools.partial(
    pl.kernel,
    out_type=jax.ShapeDtypeStruct((NROI, NB, C), jnp.float32),
    mesh=plsc.VectorSubcoreMesh(core_axis_name="c", subcore_axis_name="s"),
    scratch_types=[
        pltpu.VMEM((RPW,), jnp.float32),
        pltpu.VMEM((RPW,), jnp.float32),
        pltpu.VMEM((RPW,), jnp.float32),
        pltpu.VMEM((RPW,), jnp.float32),
        pltpu.VMEM((RPW,), jnp.float32),
        pltpu.VMEM((NCHUNK, CHUNK_ROWS), jnp.int32),
        pltpu.VMEM((NB, 16), jnp.float32),
        pltpu.VMEM((2, CHUNK_ROWS, C), jnp.float32),
        pltpu.VMEM((NB, C), jnp.float32),
        pltpu.SemaphoreType.DMA,
        pltpu.SemaphoreType.DMA,
    ],
)(_sc_body)


def kernel(feat0, feat1, feat2, feat3, rois):
    table = jnp.concatenate(
        [f.transpose(0, 2, 3, 1).reshape(-1, C)
         for f in (feat0, feat1, feat2, feat3)], axis=0)
    rp = jnp.pad(rois, ((0, NROI_PAD - rois.shape[0]), (0, 0)))
    out = _roi_align_sc(table, rp[:, 0], rp[:, 1], rp[:, 2], rp[:, 3], rp[:, 4])
    return out.reshape(NROI, OUT_SIZE, OUT_SIZE, C).transpose(0, 3, 1, 2)


# speedup vs baseline: 1.0127x; 1.0127x over previous
import functools

import jax
import jax.numpy as jnp
from jax import lax
from jax.experimental import pallas as pl
from jax.experimental.pallas import tpu as pltpu
from jax.experimental.pallas import tpu_sc as plsc

OUT_SIZE = 7
NB = OUT_SIZE * OUT_SIZE
C = 256
CV = C // 16
NC, NS = 2, 16
NW = NC * NS
RPW = 32
NROI = 1000
NROI_PAD = NW * RPW
CHUNK_BINS = 7
NCHUNK = NB // CHUNK_BINS
CHUNK_ROWS = CHUNK_BINS * 16
NBUF = 3
TOT_ROWS = 2 * (128 * 128 + 64 * 64 + 32 * 32 + 16 * 16)

_T1 = float((56.0 * (2.0 ** 1 - 1e-6)) ** 2)
_T2 = float((56.0 * (2.0 ** 2 - 1e-6)) ** 2)
_T3 = float((56.0 * (2.0 ** 3 - 1e-6)) ** 2)

_GATHER_DNUMS = lax.GatherDimensionNumbers(
    offset_dims=(), collapsed_slice_dims=(0,), start_index_map=(0,))


def _vreg_gather(vec, idx):
    return lax.gather(vec, idx[:, None], _GATHER_DNUMS, (1,),
                      mode=lax.GatherScatterMode.PROMISE_IN_BOUNDS)


def _sc_body(table, rb, rx1, ry1, rx2, ry2, out,
             rbv, rx1v, ry1v, rx2v, ry2v,
             idx_buf, w_buf, rows_buf, bins_buf, sem0, sem1, sem2):
    wid = lax.axis_index("c") * NS + lax.axis_index("s")
    base_roi = wid * RPW
    pltpu.sync_copy(rb.at[pl.ds(base_roi, RPW)], rbv)
    pltpu.sync_copy(rx1.at[pl.ds(base_roi, RPW)], rx1v)
    pltpu.sync_copy(ry1.at[pl.ds(base_roi, RPW)], ry1v)
    pltpu.sync_copy(rx2.at[pl.ds(base_roi, RPW)], rx2v)
    pltpu.sync_copy(ry2.at[pl.ds(base_roi, RPW)], ry2v)

    lane = jnp.arange(16, dtype=jnp.int32)
    dy = lane >> 3
    dx = (lane >> 2) & 1
    ty = (lane >> 1) & 1
    tx = lane & 1

    def roi_body(r, carry):
        half = r & 16
        ridx = jnp.full((16,), r & 15, jnp.int32)

        def bcast(ref):
            return _vreg_gather(ref[pl.ds(half, 16)], ridx)

        bb = bcast(rbv)
        xx1 = bcast(rx1v)
        yy1 = bcast(ry1v)
        xx2 = bcast(rx2v)
        yy2 = bcast(ry2v)

        area = (xx2 - xx1 + 1.0) * (yy2 - yy1 + 1.0)
        c1 = area >= _T1
        c2 = area >= _T2
        c3 = area >= _T3
        ss = jnp.where(c3, 0.03125, jnp.where(c2, 0.0625, jnp.where(c1, 0.125, 0.25)))
        wdim = jnp.where(c3, 16, jnp.where(c2, 32, jnp.where(c1, 64, 128)))
        basev = jnp.where(c3, 43008, jnp.where(c2, 40960, jnp.where(c1, 32768, 0)))
        basev = basev + bb.astype(jnp.int32) * (wdim * wdim)
        wm1 = wdim - 1
        wm1f = wm1.astype(jnp.float32)
        x1s = xx1 * ss
        y1s = yy1 * ss
        rw7 = jnp.maximum(xx2 * ss - x1s, 1.0) * (1.0 / OUT_SIZE)
        rh7 = jnp.maximum(yy2 * ss - y1s, 1.0) * (1.0 / OUT_SIZE)

        def row_body(i, _):
            def col_body(j, _):
                s_y = dy + 2 * i
                s_x = dx + 2 * j
                yy = y1s + (s_y.astype(jnp.float32) + 0.5) * 0.5 * rh7
                xx = x1s + (s_x.astype(jnp.float32) + 0.5) * 0.5 * rw7
                yy = jnp.clip(yy, 0.0, wm1f)
                xx = jnp.clip(xx, 0.0, wm1f)
                y0 = yy.astype(jnp.int32)
                x0 = xx.astype(jnp.int32)
                lyv = yy - y0.astype(jnp.float32)
                lxv = xx - x0.astype(jnp.float32)
                y1c = jnp.minimum(y0 + 1, wm1)
                x1c = jnp.minimum(x0 + 1, wm1)
                ysel = jnp.where(ty == 1, y1c, y0)
                xsel = jnp.where(tx == 1, x1c, x0)
                wyv = jnp.where(ty == 1, lyv, 1.0 - lyv)
                wxv = jnp.where(tx == 1, lxv, 1.0 - lxv)
                idx_buf[i, pl.ds(j * 16, 16)] = basev + ysel * wdim + xsel
                w_buf[i * OUT_SIZE + j, :] = wyv * wxv * 0.25
                return 0
            return lax.fori_loop(0, OUT_SIZE, col_body, 0)
        lax.fori_loop(0, OUT_SIZE, row_body, 0)

        sems = (sem0, sem1, sem2)

        def fire(cs):
            return pltpu.async_copy(table.at[idx_buf.at[cs]],
                                    rows_buf.at[cs % NBUF], sems[cs % NBUF])

        handles = [fire(0), fire(1)]
        for cs in range(NCHUNK):
            handles[cs].wait()
            if cs + NBUF - 1 < NCHUNK:
                handles.append(fire(cs + NBUF - 1))

            def bin_body(bi, _, cs=cs):
                binid = cs * CHUNK_BINS + bi
                wv = w_buf[binid, :]
                acc = [jnp.zeros((16,), jnp.float32) for _ in range(CV)]
                for q in range(16):
                    wq = _vreg_gather(wv, jnp.full((16,), q, jnp.int32))
                    rrow = bi * 16 + q
                    for cc in range(CV):
                        acc[cc] = acc[cc] + wq * rows_buf[cs % NBUF, rrow,
                                                          pl.ds(cc * 16, 16)]
                for cc in range(CV):
                    bins_buf[binid, pl.ds(cc * 16, 16)] = acc[cc]
                return 0
            lax.fori_loop(0, CHUNK_BINS, bin_body, 0)

        rg = r * NW + wid

        @pl.when(rg < NROI)
        def _():
            pltpu.sync_copy(bins_buf, out.at[rg])

        return carry

    lax.fori_loop(0, RPW, roi_body, 0)


_roi_align_sc = functools.partial(
    pl.kernel,
    out_type=jax.ShapeDtypeStruct((NROI, NB, C), jnp.float32),
    mesh=plsc.VectorSubcoreMesh(core_axis_name="c", subcore_axis_name="s"),
    scratch_types=[
        pltpu.VMEM((RPW,), jnp.float32),
        pltpu.VMEM((RPW,), jnp.float32),
        pltpu.VMEM((RPW,), jnp.float32),
        pltpu.VMEM((RPW,), jnp.float32),
        pltpu.VMEM((RPW,), jnp.float32),
        pltpu.VMEM((NCHUNK, CHUNK_ROWS), jnp.int32),
        pltpu.VMEM((NB, 16), jnp.float32),
        pltpu.VMEM((NBUF, CHUNK_ROWS, C), jnp.float32),
        pltpu.VMEM((NB, C), jnp.float32),
        pltpu.SemaphoreType.DMA,
        pltpu.SemaphoreType.DMA,
        pltpu.SemaphoreType.DMA,
    ],
)(_sc_body)


def kernel(feat0, feat1, feat2, feat3, rois):
    table = jnp.concatenate(
        [f.transpose(0, 2, 3, 1).reshape(-1, C)
         for f in (feat0, feat1, feat2, feat3)], axis=0)
    rp = jnp.pad(rois, ((0, NROI_PAD - rois.shape[0]), (0, 0)))
    rp = rp.reshape(RPW, NW, 5).transpose(1, 0, 2).reshape(NROI_PAD, 5)
    out = _roi_align_sc(table, rp[:, 0], rp[:, 1], rp[:, 2], rp[:, 3], rp[:, 4])
    return out.reshape(NROI, OUT_SIZE, OUT_SIZE, C).transpose(0, 3, 1, 2)


# speedup vs baseline: 1.0323x; 1.0194x over previous
import functools

import jax
import jax.numpy as jnp
from jax import lax
from jax.experimental import pallas as pl
from jax.experimental.pallas import tpu as pltpu
from jax.experimental.pallas import tpu_sc as plsc

OUT_SIZE = 7
NB = OUT_SIZE * OUT_SIZE
C = 256
CV = C // 16
NC, NS = 2, 16
NW = NC * NS
RPW = 32
NROI = 1000
NROI_PAD = NW * RPW
CHUNK_BINS = 7
NCHUNK = NB // CHUNK_BINS
CHUNK_ROWS = CHUNK_BINS * 16
NBUF = 2
TOT_ROWS = 2 * (128 * 128 + 64 * 64 + 32 * 32 + 16 * 16)

_T1 = float((56.0 * (2.0 ** 1 - 1e-6)) ** 2)
_T2 = float((56.0 * (2.0 ** 2 - 1e-6)) ** 2)
_T3 = float((56.0 * (2.0 ** 3 - 1e-6)) ** 2)

_GATHER_DNUMS = lax.GatherDimensionNumbers(
    offset_dims=(), collapsed_slice_dims=(0,), start_index_map=(0,))


def _vreg_gather(vec, idx):
    return lax.gather(vec, idx[:, None], _GATHER_DNUMS, (1,),
                      mode=lax.GatherScatterMode.PROMISE_IN_BOUNDS)


def _sc_body(table, rb, rx1, ry1, rx2, ry2, out,
             rbv, rx1v, ry1v, rx2v, ry2v,
             idx_buf, w_buf, rows_buf, bins_buf,
             sem0, sem1, out_sem):
    wid = lax.axis_index("c") * NS + lax.axis_index("s")
    base_roi = wid * RPW
    pltpu.sync_copy(rb.at[pl.ds(base_roi, RPW)], rbv)
    pltpu.sync_copy(rx1.at[pl.ds(base_roi, RPW)], rx1v)
    pltpu.sync_copy(ry1.at[pl.ds(base_roi, RPW)], ry1v)
    pltpu.sync_copy(ry2.at[pl.ds(base_roi, RPW)], ry2v)
    pltpu.sync_copy(rx2.at[pl.ds(base_roi, RPW)], rx2v)

    lane = jnp.arange(16, dtype=jnp.int32)
    dy = lane >> 3
    dx = (lane >> 2) & 1
    ty = (lane >> 1) & 1
    tx = lane & 1
    sems = (sem0, sem1)

    def phase1(rr, pp):
        ridx = jnp.full((16,), rr & 15, jnp.int32)
        half = rr & 16

        def bcast(ref):
            return _vreg_gather(ref[pl.ds(half, 16)], ridx)

        bb = bcast(rbv)
        xx1 = bcast(rx1v)
        yy1 = bcast(ry1v)
        xx2 = bcast(rx2v)
        yy2 = bcast(ry2v)

        area = (xx2 - xx1 + 1.0) * (yy2 - yy1 + 1.0)
        c1 = area >= _T1
        c2 = area >= _T2
        c3 = area >= _T3
        ss = jnp.where(c3, 0.03125,
                       jnp.where(c2, 0.0625, jnp.where(c1, 0.125, 0.25)))
        wdim = jnp.where(c3, 16, jnp.where(c2, 32, jnp.where(c1, 64, 128)))
        basev = jnp.where(c3, 43008, jnp.where(c2, 40960,
                                               jnp.where(c1, 32768, 0)))
        basev = basev + bb.astype(jnp.int32) * (wdim * wdim)
        wm1 = wdim - 1
        wm1f = wm1.astype(jnp.float32)
        x1s = xx1 * ss
        y1s = yy1 * ss
        rw7 = jnp.maximum(xx2 * ss - x1s, 1.0) * (1.0 / OUT_SIZE)
        rh7 = jnp.maximum(yy2 * ss - y1s, 1.0) * (1.0 / OUT_SIZE)

        yterm = []
        for i in range(OUT_SIZE):
            s_y = dy + 2 * i
            yyv = y1s + (s_y.astype(jnp.float32) + 0.5) * 0.5 * rh7
            yyv = jnp.clip(yyv, 0.0, wm1f)
            y0 = yyv.astype(jnp.int32)
            lyv = yyv - y0.astype(jnp.float32)
            y1c = jnp.minimum(y0 + 1, wm1)
            ysel = jnp.where(ty == 1, y1c, y0)
            wyv = jnp.where(ty == 1, lyv, 1.0 - lyv)
            yterm.append((basev + ysel * wdim, wyv * 0.25))

        def col_body(j, _):
            s_x = dx + 2 * j
            xxv = x1s + (s_x.astype(jnp.float32) + 0.5) * 0.5 * rw7
            xxv = jnp.clip(xxv, 0.0, wm1f)
            x0 = xxv.astype(jnp.int32)
            lxv = xxv - x0.astype(jnp.float32)
            x1c = jnp.minimum(x0 + 1, wm1)
            xsel = jnp.where(tx == 1, x1c, x0)
            wxv = jnp.where(tx == 1, lxv, 1.0 - lxv)
            for i in range(OUT_SIZE):
                idx_buf[pp, i, pl.ds(j * 16, 16)] = yterm[i][0] + xsel
                w_buf[pp, i * OUT_SIZE + j, :] = yterm[i][1] * wxv
            return 0
        lax.fori_loop(0, OUT_SIZE, col_body, 0)

    phase1(0, 0)

    def roi_body(r, carry):
        rp = r & 1

        def fire(cs):
            return pltpu.async_copy(table.at[idx_buf.at[rp, cs]],
                                    rows_buf.at[cs % NBUF], sems[cs % NBUF])

        handles = [fire(i) for i in range(NBUF - 1)]

        rg = r * NW + wid

        @pl.when(jnp.logical_and(r >= 2, rg - 2 * NW < NROI))
        def _():
            pltpu.make_async_copy(bins_buf.at[rp], out.at[0], out_sem).wait()

        @pl.when(r + 1 < RPW)
        def _():
            phase1(r + 1, 1 - rp)

        for cs in range(NCHUNK):
            handles[cs].wait()
            if cs + NBUF - 1 < NCHUNK:
                handles.append(fire(cs + NBUF - 1))

            def bin_body(bi, _, cs=cs):
                binid = cs * CHUNK_BINS + bi
                wv = w_buf[rp, binid, :]
                acc = [jnp.zeros((16,), jnp.float32) for _ in range(CV)]
                for q in range(16):
                    wq = _vreg_gather(wv, jnp.full((16,), q, jnp.int32))
                    rrow = bi * 16 + q
                    for cc in range(CV):
                        acc[cc] = acc[cc] + wq * rows_buf[cs % NBUF, rrow,
                                                          pl.ds(cc * 16, 16)]
                for cc in range(CV):
                    bins_buf[rp, binid, pl.ds(cc * 16, 16)] = acc[cc]
                return 0
            lax.fori_loop(0, CHUNK_BINS, bin_body, 0)

        @pl.when(rg < NROI)
        def _():
            pltpu.async_copy(bins_buf.at[rp], out.at[rg], out_sem)

        return carry

    lax.fori_loop(0, RPW, roi_body, 0)

    for k in (RPW - 2, RPW - 1):
        @pl.when(k * NW + wid < NROI)
        def _(k=k):
            pltpu.make_async_copy(bins_buf.at[k & 1], out.at[0],
                                  out_sem).wait()


_roi_align_sc = functools.partial(
    pl.kernel,
    out_type=jax.ShapeDtypeStruct((NROI, NB, C), jnp.float32),
    mesh=plsc.VectorSubcoreMesh(core_axis_name="c", subcore_axis_name="s"),
    scratch_types=[
        pltpu.VMEM((RPW,), jnp.float32),
        pltpu.VMEM((RPW,), jnp.float32),
        pltpu.VMEM((RPW,), jnp.float32),
        pltpu.VMEM((RPW,), jnp.float32),
        pltpu.VMEM((RPW,), jnp.float32),
        pltpu.VMEM((2, NCHUNK, CHUNK_ROWS), jnp.int32),
        pltpu.VMEM((2, NB, 16), jnp.float32),
        pltpu.VMEM((NBUF, CHUNK_ROWS, C), jnp.float32),
        pltpu.VMEM((2, NB, C), jnp.float32),
        pltpu.SemaphoreType.DMA,
        pltpu.SemaphoreType.DMA,
        pltpu.SemaphoreType.DMA,
    ],
)(_sc_body)


def kernel(feat0, feat1, feat2, feat3, rois):
    table = jnp.concatenate(
        [f.transpose(0, 2, 3, 1).reshape(-1, C)
         for f in (feat0, feat1, feat2, feat3)], axis=0)
    rp = jnp.pad(rois, ((0, NROI_PAD - rois.shape[0]), (0, 0)))
    rp = rp.reshape(RPW, NW, 5).transpose(1, 0, 2).reshape(NROI_PAD, 5)
    out = _roi_align_sc(table, rp[:, 0], rp[:, 1], rp[:, 2], rp[:, 3], rp[:, 4])
    return out.reshape(NROI, OUT_SIZE, OUT_SIZE, C).transpose(0, 3, 1, 2)
